# baseline (speedup 1.0000x reference)
import functools

import jax
import jax.numpy as jnp
from jax import lax
from jax.experimental import pallas as pl
from jax.experimental.pallas import tpu as pltpu
from jax.experimental.pallas import tpu_sc as plsc

B = 16384
NUM = 256
F = 26
V = 100000
D = 64
H = 1024

FC = 13
NCHUNK = F // FC

VB = 8192
NJ = (V + VB - 1) // VB
PF = NJ * (VB // 4)
PR = 2 * PF

RC = B * FC
NC = 2
NS = 16
NW = NC * NS
SL = 128
SPW = RC // (NW * SL)
SPC = 4
CR = SPC * SL
CHUNKS = SPW // SPC


def _repack_body(tin_ref, out_ref):
    x = tin_ref[0]
    y = jnp.swapaxes(x.astype(jnp.bfloat16), 0, 1)
    z = pltpu.bitcast(y, jnp.int32)
    out_ref[0] = jnp.concatenate([z[0 : VB // 4], z[VB // 4 : VB // 2]], axis=1)


def _tc_repack(tbl_t, k):
    f0 = k * FC
    return pl.pallas_call(
        _repack_body,
        grid=(FC, NJ),
        in_specs=[pl.BlockSpec((1, D, VB), lambda f, j: (f0 + f, 0, j))],
        out_specs=pl.BlockSpec((1, VB // 4, 2 * D), lambda f, j: (f, j, 0)),
        out_shape=jax.ShapeDtypeStruct((FC, PF, 2 * D), jnp.int32),
    )(tbl_t)


def _sc_gather(tbl2d, idx2d):
    mesh = plsc.VectorSubcoreMesh(core_axis_name="c", subcore_axis_name="s")

    @functools.partial(
        pl.kernel,
        mesh=mesh,
        compiler_params=pltpu.CompilerParams(use_tc_tiling_on_sc=False),
        out_type=jax.ShapeDtypeStruct((RC, D), jnp.int32),
        scratch_types=[
            pltpu.VMEM((SPC, SL), jnp.int32),
            pltpu.VMEM((CR, D), jnp.int32),
            pltpu.SemaphoreType.DMA,
        ],
    )
    def k(tbl_hbm, idx_hbm, out_hbm, idxc_v, rows_v, sem):
        wid = lax.axis_index("s") * NC + lax.axis_index("c")
        slice_base = wid * SPW
        row_base = slice_base * SL

        def chunk_body(c, carry):
            s0 = slice_base + c * SPC
            pltpu.sync_copy(idx_hbm.at[pl.ds(s0, SPC)], idxc_v)
            copies = [
                pltpu.async_copy(
                    tbl_hbm.at[idxc_v.at[j]],
                    rows_v.at[pl.ds(j * SL, SL)],
                    sem,
                )
                for j in range(SPC)
            ]
            for cp in copies:
                cp.wait()
            pltpu.sync_copy(
                rows_v, out_hbm.at[pl.ds(row_base + c * CR, CR)]
            )
            return carry

        lax.fori_loop(0, CHUNKS, chunk_body, 0)

    return k(tbl2d, idx2d)


def _mlp_body(
    xnL_ref, xnH_ref, embA_ref, embB_ref,
    sALo_ref, sAHi_ref, sBLo_ref, sBHi_ref,
    w1a_ref, w1b_ref, b1_ref, w2t_ref, b2_ref, out_ref,
):
    xn2 = jnp.concatenate([xnL_ref[...], xnH_ref[...]], axis=0)
    h = jnp.dot(
        xn2.astype(jnp.bfloat16), w1a_ref[...], preferred_element_type=jnp.float32
    )
    bb2 = xn2.shape[0]
    rep = (
        lax.broadcasted_iota(jnp.int32, (F, F * D), 1) // D
        == lax.broadcasted_iota(jnp.int32, (F, F * D), 0)
    ).astype(jnp.float32)
    sxc = jnp.concatenate(
        [
            jnp.concatenate([sALo_ref[...], sAHi_ref[...]], axis=0),
            jnp.concatenate([sBLo_ref[...], sBHi_ref[...]], axis=0),
        ],
        axis=1,
    ).astype(jnp.float32)
    sw = jnp.dot(sxc, rep, preferred_element_type=jnp.float32)
    ew = jnp.concatenate(
        [
            jnp.concatenate(
                [emb_ref[f][:, 0:D], emb_ref[f][:, D : 2 * D]], axis=0
            )
            for emb_ref in (embA_ref, embB_ref)
            for f in range(FC)
        ],
        axis=1,
    )
    bits = ew << sw.astype(jnp.int32)
    e = lax.bitcast_convert_type(bits, jnp.float32).astype(jnp.bfloat16)
    h = h + jnp.dot(e, w1b_ref[...], preferred_element_type=jnp.float32)
    h = jnp.maximum(h + b1_ref[...], 0.0)
    s = jax.nn.sigmoid(
        jnp.sum(h * w2t_ref[...], axis=1, keepdims=True) + b2_ref[0]
    )
    out_ref[0] = s[0 : bb2 // 2]
    out_ref[1] = s[bb2 // 2 : bb2]


def _tc_mlp(x_num, embs, sixes, w1a, w1b, b1r, w2r, b2r, bb=512):
    nj = B // (2 * bb)
    return pl.pallas_call(
        _mlp_body,
        grid=(nj,),
        in_specs=[
            pl.BlockSpec((bb, NUM), lambda i: (i, 0)),
            pl.BlockSpec((bb, NUM), lambda i: (i + nj, 0)),
            pl.BlockSpec((FC, bb, 2 * D), lambda i: (0, i, 0)),
            pl.BlockSpec((FC, bb, 2 * D), lambda i: (0, i, 0)),
            pl.BlockSpec((bb, FC), lambda i: (i, 0)),
            pl.BlockSpec((bb, FC), lambda i: (i + nj, 0)),
            pl.BlockSpec((bb, FC), lambda i: (i, 0)),
            pl.BlockSpec((bb, FC), lambda i: (i + nj, 0)),
            pl.BlockSpec((NUM, H), lambda i: (0, 0)),
            pl.BlockSpec((F * D, H), lambda i: (0, 0)),
            pl.BlockSpec((1, H), lambda i: (0, 0)),
            pl.BlockSpec((1, H), lambda i: (0, 0)),
            pl.BlockSpec(memory_space=pltpu.SMEM),
        ],
        out_specs=pl.BlockSpec((2, bb, 1), lambda i: (0, i, 0)),
        out_shape=jax.ShapeDtypeStruct((2, B // 2, 1), jnp.float32),
    )(
        x_num, x_num, embs[0], embs[1],
        sixes[0], sixes[0], sixes[1], sixes[1],
        w1a, w1b, b1r, w2r, b2r,
    )


def kernel(x_num, x_cat, tables, W1, b1, W2, b2):
    tbl_t = jnp.transpose(tables, (0, 2, 1))
    xc = jnp.transpose(x_cat.astype(jnp.int32), (1, 0))
    fcol = lax.broadcasted_iota(jnp.int32, (FC, B // 2, 2), 0)
    embs, sixes = [], []
    for k in range(NCHUNK):
        xck = xc[k * FC : (k + 1) * FC]
        xcp = jnp.stack([xck[:, : B // 2], xck[:, B // 2 :]], axis=2)
        zrow = (xcp & (VB - 1)) >> 1
        rows = (
            fcol * PR
            + (xcp >> 13) * (VB // 2)
            + ((zrow & (VB // 4 - 1)) << 1)
            + (zrow >> 11)
        )
        sixes.append(jnp.transpose((1 - (xck & 1)) << 4))
        tbl2d = _tc_repack(tbl_t, k).reshape(FC * PR, D)
        g = _sc_gather(tbl2d, rows.reshape(RC // SL, SL))
        embs.append(g.reshape(FC, B // 2, 2 * D))
        del xcp, zrow, rows
    out2 = _tc_mlp(
        x_num,
        embs,
        sixes,
        W1[:NUM].astype(jnp.bfloat16),
        W1[NUM:].astype(jnp.bfloat16),
        b1.reshape(1, H),
        W2.reshape(1, H),
        b2.reshape(1),
    )
    return out2.reshape(B, 1)


# speedup vs baseline: 1.0506x; 1.0506x over previous
import functools

import jax
import jax.numpy as jnp
from jax import lax
from jax.experimental import pallas as pl
from jax.experimental.pallas import tpu as pltpu
from jax.experimental.pallas import tpu_sc as plsc

B = 16384
NUM = 256
F = 26
V = 100000
D = 64
H = 1024

FC = 13
NCHUNK = F // FC

VB = 8192
NJ = (V + VB - 1) // VB
PF = NJ * (VB // 4)

RC = B * FC
NC = 2
NS = 16
NW = NC * NS
SL = 128
SPW = RC // (NW * SL)
SPC = 4
CR = SPC * SL
CHUNKS = SPW // SPC


def _repack_body(tin_ref, out_ref):
    x = tin_ref[0]
    y = jnp.swapaxes(x.astype(jnp.bfloat16), 0, 1)
    z = pltpu.bitcast(y, jnp.int32)
    out_ref[0] = jnp.concatenate([z[0 : VB // 4], z[VB // 4 : VB // 2]], axis=1)


def _tc_repack(tbl_t, k):
    f0 = k * FC
    return pl.pallas_call(
        _repack_body,
        grid=(FC, NJ),
        in_specs=[pl.BlockSpec((1, D, VB), lambda f, j: (f0 + f, 0, j))],
        out_specs=pl.BlockSpec((1, VB // 4, 2 * D), lambda f, j: (f, j, 0)),
        out_shape=jax.ShapeDtypeStruct((FC, PF, 2 * D), jnp.int32),
    )(tbl_t)


def _sc_gather(tbl2d, idx2d):
    mesh = plsc.VectorSubcoreMesh(core_axis_name="c", subcore_axis_name="s")

    @functools.partial(
        pl.kernel,
        mesh=mesh,
        compiler_params=pltpu.CompilerParams(use_tc_tiling_on_sc=True),
        out_type=jax.ShapeDtypeStruct((RC, 2 * D), jnp.int32),
        scratch_types=[
            pltpu.VMEM((SPC, SL), jnp.int32),
            pltpu.VMEM((CR, 2 * D), jnp.int32),
            pltpu.SemaphoreType.DMA,
        ],
    )
    def k(tbl_hbm, idx_hbm, out_hbm, idxc_v, rows_v, sem):
        wid = lax.axis_index("s") * NC + lax.axis_index("c")
        slice_base = wid * SPW
        row_base = slice_base * SL

        def chunk_body(c, carry):
            s0 = slice_base + c * SPC
            pltpu.sync_copy(idx_hbm.at[pl.ds(s0, SPC)], idxc_v)
            copies = [
                pltpu.async_copy(
                    tbl_hbm.at[idxc_v.at[j]],
                    rows_v.at[pl.ds(j * SL, SL)],
                    sem,
                )
                for j in range(SPC)
            ]
            for cp in copies:
                cp.wait()
            pltpu.sync_copy(
                rows_v, out_hbm.at[pl.ds(row_base + c * CR, CR)]
            )
            return carry

        lax.fori_loop(0, CHUNKS, chunk_body, 0)

    return k(tbl2d, idx2d)


def _mlp_body(
    xn_ref, embA_ref, embB_ref, halfA_ref, halfB_ref, sixA_ref, sixB_ref,
    w1a_ref, w1b_ref, b1_ref, w2t_ref, b2_ref, out_ref,
):
    h = jnp.dot(
        xn_ref[...].astype(jnp.bfloat16),
        w1a_ref[...],
        preferred_element_type=jnp.float32,
    )
    bb = xn_ref.shape[0]
    rep = (
        lax.broadcasted_iota(jnp.int32, (F, F * D), 1) // D
        == lax.broadcasted_iota(jnp.int32, (F, F * D), 0)
    ).astype(jnp.float32)
    ctl = jnp.concatenate(
        [halfA_ref[...], halfB_ref[...]], axis=1
    ).astype(jnp.float32)
    sxc = jnp.concatenate(
        [sixA_ref[...], sixB_ref[...]], axis=1
    ).astype(jnp.float32)
    hw = jnp.dot(ctl, rep, preferred_element_type=jnp.float32)
    sw = jnp.dot(sxc, rep, preferred_element_type=jnp.float32)
    lw = jnp.concatenate(
        [emb_ref[f][:, 0:D] for emb_ref in (embA_ref, embB_ref) for f in range(FC)],
        axis=1,
    )
    rw = jnp.concatenate(
        [emb_ref[f][:, D : 2 * D] for emb_ref in (embA_ref, embB_ref) for f in range(FC)],
        axis=1,
    )
    s64 = jnp.where(hw > 0.5, rw, lw)
    bits = s64 << sw.astype(jnp.int32)
    e = lax.bitcast_convert_type(bits, jnp.float32).astype(jnp.bfloat16)
    h = h + jnp.dot(e, w1b_ref[...], preferred_element_type=jnp.float32)
    h = jnp.maximum(h + b1_ref[...], 0.0)
    s = jnp.sum(h * w2t_ref[...], axis=1, keepdims=True) + b2_ref[0]
    out_ref[...] = jax.nn.sigmoid(s)


def _tc_mlp(x_num, embs, halves, sixes, w1a, w1b, b1r, w2r, b2r, bb=1024):
    return pl.pallas_call(
        _mlp_body,
        grid=(B // bb,),
        in_specs=[
            pl.BlockSpec((bb, NUM), lambda i: (i, 0)),
            pl.BlockSpec((FC, bb, 2 * D), lambda i: (0, i, 0)),
            pl.BlockSpec((FC, bb, 2 * D), lambda i: (0, i, 0)),
            pl.BlockSpec((bb, FC), lambda i: (i, 0)),
            pl.BlockSpec((bb, FC), lambda i: (i, 0)),
            pl.BlockSpec((bb, FC), lambda i: (i, 0)),
            pl.BlockSpec((bb, FC), lambda i: (i, 0)),
            pl.BlockSpec((NUM, H), lambda i: (0, 0)),
            pl.BlockSpec((F * D, H), lambda i: (0, 0)),
            pl.BlockSpec((1, H), lambda i: (0, 0)),
            pl.BlockSpec((1, H), lambda i: (0, 0)),
            pl.BlockSpec(memory_space=pltpu.SMEM),
        ],
        out_specs=pl.BlockSpec((bb, 1), lambda i: (i, 0)),
        out_shape=jax.ShapeDtypeStruct((B, 1), jnp.float32),
    )(
        x_num, embs[0], embs[1], halves[0], halves[1], sixes[0], sixes[1],
        w1a, w1b, b1r, w2r, b2r,
    )


def kernel(x_num, x_cat, tables, W1, b1, W2, b2):
    tbl_t = jnp.transpose(tables, (0, 2, 1))
    xc = jnp.transpose(x_cat.astype(jnp.int32), (1, 0))
    fcol = lax.broadcasted_iota(jnp.int32, (FC, B), 0)
    embs, halves, sixes = [], [], []
    for k in range(NCHUNK):
        xck = xc[k * FC : (k + 1) * FC]
        zrow = (xck & (VB - 1)) >> 1
        rows = fcol * PF + (xck >> 13) * (VB // 4) + (zrow & (VB // 4 - 1))
        halves.append(jnp.transpose(zrow >> 11))
        sixes.append(jnp.transpose((1 - (xck & 1)) << 4))
        tbl2d = _tc_repack(tbl_t, k).reshape(FC * PF, 2 * D)
        g = _sc_gather(tbl2d, rows.reshape(RC // SL, SL))
        embs.append(g.reshape(FC, B, 2 * D))
    out = _tc_mlp(
        x_num,
        embs,
        halves,
        sixes,
        W1[:NUM].astype(jnp.bfloat16),
        W1[NUM:].astype(jnp.bfloat16),
        b1.reshape(1, H),
        W2.reshape(1, H),
        b2.reshape(1),
    )
    return out
